# Initial kernel scaffold; baseline (speedup 1.0000x reference)
import jax
import jax.numpy as jnp
from jax.experimental import pallas as pl


def kernel(feature, edge_index, W, b):
    raise NotImplementedError("write your pallas kernel here")


# speedup vs baseline: 7.7996x; 7.7996x over previous
import functools

import jax
import jax.numpy as jnp
from jax import lax
from jax.experimental import pallas as pl
from jax.experimental.pallas import tpu as pltpu
from jax.experimental.pallas import tpu_sc as plsc

N_NODES = 10000
N_EDGES = 320000
D = 128

NC = 2
NS = 16
NW = NC * NS
EPW = N_EDGES // NW
CHUNK = 80
NCHUNK = EPW // CHUNK
N_PAD = 10240
SROWS = N_PAD // NS
ZROWS = 8
DEGW = 16
IDX_SHIFT = 14


def _sc_edge_kernel(feature, packed_idx):
    mesh = plsc.VectorSubcoreMesh(
        core_axis_name="c", subcore_axis_name="s", num_cores=NC, num_subcores=NS
    )

    @functools.partial(
        pl.kernel,
        mesh=mesh,
        compiler_params=pltpu.CompilerParams(use_tc_tiling_on_sc=False, needs_layout_passes=False),
        out_type=(
            jax.ShapeDtypeStruct((NC, N_PAD, D), jnp.float32),
            jax.ShapeDtypeStruct((NW, N_PAD), jnp.float32),
        ),
        scratch_types=[
            pltpu.VMEM_SHARED((N_PAD, D), jnp.float32),
            pltpu.VMEM((128, CHUNK), jnp.int32),
            pltpu.VMEM((128,), jnp.int32),
            pltpu.VMEM((CHUNK,), jnp.int32),
            pltpu.VMEM((CHUNK,), jnp.int32),
            pltpu.VMEM((CHUNK, D), jnp.float32),
            pltpu.VMEM((CHUNK, D), jnp.float32),
            pltpu.VMEM((ZROWS, D), jnp.float32),
            pltpu.VMEM((N_PAD,), jnp.float32),
            pltpu.SemaphoreType.DMA,
            pltpu.SemaphoreType.DMA,
        ],
    )
    def k(feat_hbm, pk_hbm, out_hbm, deg_hbm,
          acc, pidx, rv, sbuf, dbuf, abuf, bbuf, zbuf, degp, sem1, sem2):
        c = lax.axis_index("c")
        s = lax.axis_index("s")
        wid = c * NS + s

        zv = jnp.zeros((16,), jnp.float32)

        def fill_z(r, _):
            for t in range(D // 16):
                zbuf[r, pl.ds(t * 16, 16)] = zv
            return 0
        lax.fori_loop(0, ZROWS, fill_z, 0)

        def fill_dz(r, _):
            degp[pl.ds(r * 16, 16)] = zv
            return 0
        lax.fori_loop(0, N_PAD // 16, fill_dz, 0)

        base = s * SROWS
        for kk in range(SROWS // ZROWS):
            pltpu.sync_copy(zbuf, acc.at[pl.ds(base + kk * ZROWS, ZROWS)])
        plsc.subcore_barrier()

        lanes = lax.iota(jnp.int32, 16)
        for t in range(128 // 16):
            rv[pl.ds(t * 16, 16)] = wid * NCHUNK + t * 16 + lanes
        pltpu.async_copy(pk_hbm.at[rv], pidx, sem1).wait()
        mask = jnp.full((16,), (1 << IDX_SHIFT) - 1, jnp.int32)

        onev = jnp.full((16,), 1.0, jnp.float32)

        def chunk_body(j, _):
            for t in range(CHUNK // 16):
                sl = pl.ds(t * 16, 16)
                v = pidx[j, sl]
                sbuf[sl] = v & mask
                dbuf[sl] = lax.shift_right_logical(v, IDX_SHIFT)

            cp1 = pltpu.async_copy(feat_hbm.at[sbuf], abuf, sem1)
            cp2 = pltpu.async_copy(feat_hbm.at[dbuf], bbuf, sem2)
            cp1.wait()
            cp2.wait()

            def mul_row(r, _):
                for t in range(D // 16):
                    sl = pl.ds(t * 16, 16)
                    abuf[r, sl] = abuf[r, sl] * bbuf[r, sl]
                return 0
            lax.fori_loop(0, CHUNK, mul_row, 0)

            for t in range(CHUNK // 16):
                plsc.addupdate_scatter(degp, [dbuf[pl.ds(t * 16, 16)]], onev)

            pltpu.sync_copy(abuf, acc.at[dbuf], add=True)
            return 0
        lax.fori_loop(0, NCHUNK, chunk_body, 0)

        plsc.subcore_barrier()

        pltpu.sync_copy(acc.at[pl.ds(base, SROWS)],
                        out_hbm.at[c, pl.ds(base, SROWS)])
        pltpu.sync_copy(degp, deg_hbm.at[wid])

    return k(feature, packed_idx)


def _tc_finish_kernel(parts, degp, wt, b2):
    R = 1024

    def body(p_ref, d_ref, wt_ref, b_ref, o_ref):
        ssum = p_ref[0] + p_ref[1]
        dsum = jnp.sum(d_ref[...], axis=0)[:, None]
        x = ssum / jnp.maximum(dsum, 1.0)
        o_ref[...] = (
            jnp.dot(x, wt_ref[...], preferred_element_type=jnp.float32) + b_ref[...]
        )

    return pl.pallas_call(
        body,
        grid=((N_NODES + R - 1) // R,),
        in_specs=[
            pl.BlockSpec((NC, R, D), lambda i: (0, i, 0)),
            pl.BlockSpec((NW, R), lambda i: (0, i)),
            pl.BlockSpec((D, D), lambda i: (0, 0)),
            pl.BlockSpec((1, D), lambda i: (0, 0)),
        ],
        out_specs=pl.BlockSpec((R, D), lambda i: (i, 0)),
        out_shape=jax.ShapeDtypeStruct((N_NODES, D), jnp.float32),
    )(parts, degp, wt, b2)


def kernel(feature, edge_index, W, b):
    ei = edge_index.astype(jnp.int32)
    packed = (ei[0] | (ei[1] << IDX_SHIFT)).reshape(NW * NCHUNK, CHUNK)
    packed = jnp.pad(packed, ((0, 4096 - NW * NCHUNK), (0, 0)))
    parts, degp = _sc_edge_kernel(feature, packed)
    return _tc_finish_kernel(parts, degp, W.T, b.reshape(1, D))


# speedup vs baseline: 11.6125x; 1.4889x over previous
import functools

import jax
import jax.numpy as jnp
from jax import lax
from jax.experimental import pallas as pl
from jax.experimental.pallas import tpu as pltpu
from jax.experimental.pallas import tpu_sc as plsc

N_NODES = 10000
N_EDGES = 320000
D = 128

NC = 2
NS = 16
NW = NC * NS
EPW = N_EDGES // NW
CHUNK = 80
NCHUNK = EPW // CHUNK
N_PAD = 10240
SROWS = N_PAD // NS
ZROWS = 8
IDX_SHIFT = 14


def _sc_edge_kernel(feature, packed_idx):
    mesh = plsc.VectorSubcoreMesh(
        core_axis_name="c", subcore_axis_name="s", num_cores=NC, num_subcores=NS
    )

    @functools.partial(
        pl.kernel,
        mesh=mesh,
        compiler_params=pltpu.CompilerParams(use_tc_tiling_on_sc=False, needs_layout_passes=False),
        out_type=(
            jax.ShapeDtypeStruct((NC, N_PAD, D), jnp.float32),
            jax.ShapeDtypeStruct((NW, N_PAD), jnp.float32),
        ),
        scratch_types=[
            pltpu.VMEM_SHARED((N_PAD, D), jnp.float32),
            pltpu.VMEM((128, CHUNK), jnp.int32),
            pltpu.VMEM((128,), jnp.int32),
            pltpu.VMEM((CHUNK,), jnp.int32),
            pltpu.VMEM((CHUNK,), jnp.int32),
            pltpu.VMEM((CHUNK, D), jnp.float32),
            pltpu.VMEM((CHUNK, D), jnp.float32),
            pltpu.VMEM((ZROWS, D), jnp.float32),
            pltpu.VMEM((N_PAD,), jnp.float32),
            pltpu.SemaphoreType.DMA,
            pltpu.SemaphoreType.DMA,
        ],
    )
    def k(feat_hbm, pk_hbm, out_hbm, deg_hbm,
          acc, pidx, rv, sbuf, dbuf, abuf, bbuf, zbuf, degp, sem1, sem2):
        c = lax.axis_index("c")
        s = lax.axis_index("s")
        wid = c * NS + s

        zv = jnp.zeros((16,), jnp.float32)

        def fill_z(r, _):
            for t in range(D // 16):
                zbuf[r, pl.ds(t * 16, 16)] = zv
            return 0
        lax.fori_loop(0, ZROWS, fill_z, 0)

        def fill_dz(r, _):
            degp[pl.ds(r * 16, 16)] = zv
            return 0
        lax.fori_loop(0, N_PAD // 16, fill_dz, 0)

        base = s * SROWS
        for kk in range(SROWS // ZROWS):
            pltpu.sync_copy(zbuf, acc.at[pl.ds(base + kk * ZROWS, ZROWS)])
        plsc.subcore_barrier()

        lanes = lax.iota(jnp.int32, 16)
        for t in range(128 // 16):
            rv[pl.ds(t * 16, 16)] = wid * NCHUNK + t * 16 + lanes
        pltpu.async_copy(pk_hbm.at[rv], pidx, sem1).wait()
        mask = jnp.full((16,), (1 << IDX_SHIFT) - 1, jnp.int32)

        onev = jnp.full((16,), 1.0, jnp.float32)

        def unpack(j):
            for t in range(CHUNK // 16):
                sl = pl.ds(t * 16, 16)
                v = pidx[j, sl]
                sbuf[sl] = v & mask
                dbuf[sl] = lax.shift_right_logical(v, IDX_SHIFT)

        def scatter_deg(j):
            for t in range(CHUNK // 16):
                plsc.addupdate_scatter(degp, [dbuf[pl.ds(t * 16, 16)]], onev)

        def chunk_body(j, _):
            unpack(j)
            pltpu.async_copy(feat_hbm.at[sbuf], abuf, sem1).wait()
            scatter_deg(j)
            pltpu.sync_copy(abuf, acc.at[dbuf], add=True)
            return 0
        lax.fori_loop(0, NCHUNK, chunk_body, 0)

        plsc.subcore_barrier()

        pltpu.sync_copy(acc.at[pl.ds(base, SROWS)],
                        out_hbm.at[c, pl.ds(base, SROWS)])
        pltpu.sync_copy(degp, deg_hbm.at[wid])

    return k(feature, packed_idx)


def _tc_finish_kernel(parts, degp, feature, wt, b2):
    R = 1024

    def body(p_ref, d_ref, f_ref, wt_ref, b_ref, o_ref):
        ssum = p_ref[0] + p_ref[1]
        dsum = jnp.sum(d_ref[...], axis=0)[:, None]
        x = (f_ref[...] * ssum) / jnp.maximum(dsum, 1.0)
        o_ref[...] = (
            jnp.dot(x, wt_ref[...], preferred_element_type=jnp.float32) + b_ref[...]
        )

    return pl.pallas_call(
        body,
        grid=((N_NODES + R - 1) // R,),
        in_specs=[
            pl.BlockSpec((NC, R, D), lambda i: (0, i, 0)),
            pl.BlockSpec((NW, R), lambda i: (0, i)),
            pl.BlockSpec((R, D), lambda i: (i, 0)),
            pl.BlockSpec((D, D), lambda i: (0, 0)),
            pl.BlockSpec((1, D), lambda i: (0, 0)),
        ],
        out_specs=pl.BlockSpec((R, D), lambda i: (i, 0)),
        out_shape=jax.ShapeDtypeStruct((N_NODES, D), jnp.float32),
    )(parts, degp, feature, wt, b2)


def kernel(feature, edge_index, W, b):
    ei = edge_index.astype(jnp.int32)
    packed = (ei[0] | (ei[1] << IDX_SHIFT)).reshape(NW * NCHUNK, CHUNK)
    packed = jnp.pad(packed, ((0, 4096 - NW * NCHUNK), (0, 0)))
    parts, degp = _sc_edge_kernel(feature, packed)
    return _tc_finish_kernel(parts, degp, feature, W.T, b.reshape(1, D))


# speedup vs baseline: 17.8000x; 1.5328x over previous
import functools

import jax
import jax.numpy as jnp
from jax import lax
from jax.experimental import pallas as pl
from jax.experimental.pallas import tpu as pltpu
from jax.experimental.pallas import tpu_sc as plsc

N_NODES = 10000
N_EDGES = 320000
D = 128

NC = 2
NS = 16
NW = NC * NS
EPW = N_EDGES // NW
CHUNK = 80
NCHUNK = EPW // CHUNK
N_PAD = 10240
SROWS = N_PAD // NS
ZROWS = 8
IDX_SHIFT = 14


def _sc_edge_kernel(feature, packed_idx):
    mesh = plsc.VectorSubcoreMesh(
        core_axis_name="c", subcore_axis_name="s", num_cores=NC, num_subcores=NS
    )

    @functools.partial(
        pl.kernel,
        mesh=mesh,
        compiler_params=pltpu.CompilerParams(use_tc_tiling_on_sc=False, needs_layout_passes=False),
        out_type=(
            jax.ShapeDtypeStruct((NC, N_PAD, D), jnp.float32),
            jax.ShapeDtypeStruct((NW, N_PAD), jnp.float32),
        ),
        scratch_types=[
            pltpu.VMEM_SHARED((N_PAD, D), jnp.float32),
            pltpu.VMEM((128, CHUNK), jnp.int32),
            pltpu.VMEM((128,), jnp.int32),
            pltpu.VMEM((2, CHUNK), jnp.int32),
            pltpu.VMEM((2, CHUNK), jnp.int32),
            pltpu.VMEM((CHUNK, D), jnp.float32),
            pltpu.VMEM((CHUNK, D), jnp.float32),
            pltpu.VMEM((ZROWS, D), jnp.float32),
            pltpu.VMEM((N_PAD,), jnp.float32),
            pltpu.SemaphoreType.DMA,
            pltpu.SemaphoreType.DMA,
        ],
    )
    def k(feat_hbm, pk_hbm, out_hbm, deg_hbm,
          acc, pidx, rv, sb, db, abuf, bbuf, zbuf, degp, sem1, sem2):
        c = lax.axis_index("c")
        s = lax.axis_index("s")
        wid = c * NS + s

        zv = jnp.zeros((16,), jnp.float32)

        def fill_z(r, _):
            for t in range(D // 16):
                zbuf[r, pl.ds(t * 16, 16)] = zv
            return 0
        lax.fori_loop(0, ZROWS, fill_z, 0)

        def fill_dz(r, _):
            degp[pl.ds(r * 16, 16)] = zv
            return 0
        lax.fori_loop(0, N_PAD // 16, fill_dz, 0)

        base = s * SROWS
        for kk in range(SROWS // ZROWS):
            pltpu.sync_copy(zbuf, acc.at[pl.ds(base + kk * ZROWS, ZROWS)])
        plsc.subcore_barrier()

        lanes = lax.iota(jnp.int32, 16)
        for t in range(128 // 16):
            rv[pl.ds(t * 16, 16)] = wid * NCHUNK + t * 16 + lanes
        pltpu.async_copy(pk_hbm.at[rv], pidx, sem1).wait()
        mask = jnp.full((16,), (1 << IDX_SHIFT) - 1, jnp.int32)

        onev = jnp.full((16,), 1.0, jnp.float32)

        def unpack(j, slot):
            for t in range(CHUNK // 16):
                sl = pl.ds(t * 16, 16)
                v = pidx[j, sl]
                sb[slot, sl] = v & mask
                db[slot, sl] = lax.shift_right_logical(v, IDX_SHIFT)

        def scatter_deg(slot):
            for t in range(CHUNK // 16):
                plsc.addupdate_scatter(degp, [db[slot, pl.ds(t * 16, 16)]], onev)

        def halfstep(j_next, buf_cur, sem_cur, slot_cur, buf_nxt, sem_nxt, slot_nxt):
            unpack(j_next, slot_nxt)
            pltpu.async_copy(feat_hbm.at[sb.at[slot_nxt]], buf_nxt, sem_nxt)
            pltpu.make_async_copy(feat_hbm.at[sb.at[slot_cur]], buf_cur, sem_cur).wait()
            scatter_deg(slot_cur)
            pltpu.sync_copy(buf_cur, acc.at[db.at[slot_cur]], add=True)

        unpack(0, 0)
        pltpu.async_copy(feat_hbm.at[sb.at[0]], abuf, sem1)

        def pair_body(i, _):
            halfstep(2 * i + 1, abuf, sem1, 0, bbuf, sem2, 1)
            halfstep(2 * i + 2, bbuf, sem2, 1, abuf, sem1, 0)
            return 0
        lax.fori_loop(0, (NCHUNK - 1) // 2, pair_body, 0)

        pltpu.make_async_copy(feat_hbm.at[sb.at[0]], abuf, sem1).wait()
        scatter_deg(0)
        pltpu.sync_copy(abuf, acc.at[db.at[0]], add=True)

        plsc.subcore_barrier()

        pltpu.sync_copy(acc.at[pl.ds(base, SROWS)],
                        out_hbm.at[c, pl.ds(base, SROWS)])
        pltpu.sync_copy(degp, deg_hbm.at[wid])

    return k(feature, packed_idx)


def _tc_finish_kernel(parts, degp, feature, wt, b2):
    R = 1024

    def body(p_ref, d_ref, f_ref, wt_ref, b_ref, o_ref):
        ssum = p_ref[0] + p_ref[1]
        dsum = jnp.sum(d_ref[...], axis=0)[:, None]
        x = (f_ref[...] * ssum) / jnp.maximum(dsum, 1.0)
        o_ref[...] = (
            jnp.dot(x, wt_ref[...], preferred_element_type=jnp.float32) + b_ref[...]
        )

    return pl.pallas_call(
        body,
        grid=((N_NODES + R - 1) // R,),
        in_specs=[
            pl.BlockSpec((NC, R, D), lambda i: (0, i, 0)),
            pl.BlockSpec((NW, R), lambda i: (0, i)),
            pl.BlockSpec((R, D), lambda i: (i, 0)),
            pl.BlockSpec((D, D), lambda i: (0, 0)),
            pl.BlockSpec((1, D), lambda i: (0, 0)),
        ],
        out_specs=pl.BlockSpec((R, D), lambda i: (i, 0)),
        out_shape=jax.ShapeDtypeStruct((N_NODES, D), jnp.float32),
    )(parts, degp, feature, wt, b2)


def kernel(feature, edge_index, W, b):
    ei = edge_index.astype(jnp.int32)
    packed = (ei[0] | (ei[1] << IDX_SHIFT)).reshape(NW * NCHUNK, CHUNK)
    packed = jnp.pad(packed, ((0, 4096 - NW * NCHUNK), (0, 0)))
    parts, degp = _sc_edge_kernel(feature, packed)
    return _tc_finish_kernel(parts, degp, feature, W.T, b.reshape(1, D))


# speedup vs baseline: 19.3685x; 1.0881x over previous
import functools

import jax
import jax.numpy as jnp
from jax import lax
from jax.experimental import pallas as pl
from jax.experimental.pallas import tpu as pltpu
from jax.experimental.pallas import tpu_sc as plsc

N_NODES = 10000
N_EDGES = 320000
D = 128

NC = 2
NS = 16
NW = NC * NS
EPW = N_EDGES // NW
CHUNK = 80
NCHUNK = EPW // CHUNK
N_PAD = 10240
SROWS = N_PAD // NS
ZROWS = 8
IDX_SHIFT = 14


def _sc_edge_kernel(feature, ei):
    mesh = plsc.VectorSubcoreMesh(
        core_axis_name="c", subcore_axis_name="s", num_cores=NC, num_subcores=NS
    )

    @functools.partial(
        pl.kernel,
        mesh=mesh,
        compiler_params=pltpu.CompilerParams(use_tc_tiling_on_sc=False, needs_layout_passes=False),
        out_type=(
            jax.ShapeDtypeStruct((NC, N_PAD, D), jnp.float32),
            jax.ShapeDtypeStruct((NW, N_PAD), jnp.float32),
        ),
        scratch_types=[
            pltpu.VMEM_SHARED((N_PAD, D), jnp.float32),
            pltpu.VMEM((4, CHUNK), jnp.int32),
            pltpu.VMEM((4, CHUNK), jnp.int32),
            pltpu.VMEM((CHUNK, D), jnp.float32),
            pltpu.VMEM((CHUNK, D), jnp.float32),
            pltpu.VMEM((ZROWS, D), jnp.float32),
            pltpu.VMEM((N_PAD,), jnp.float32),
            pltpu.SemaphoreType.DMA,
            pltpu.SemaphoreType.DMA,
            pltpu.SemaphoreType.DMA,
            pltpu.SemaphoreType.DMA,
        ],
    )
    def k(feat_hbm, ei_hbm, out_hbm, deg_hbm,
          acc, sb, db, abuf, bbuf, zbuf, degp, sem1, sem2, semi1, semi2):
        c = lax.axis_index("c")
        s = lax.axis_index("s")
        wid = c * NS + s

        zv = jnp.zeros((16,), jnp.float32)

        def fill_z(r, _):
            for t in range(D // 16):
                zbuf[r, pl.ds(t * 16, 16)] = zv
            return 0
        lax.fori_loop(0, ZROWS, fill_z, 0)

        def fill_dz(r, _):
            degp[pl.ds(r * 16, 16)] = zv
            return 0
        lax.fori_loop(0, N_PAD // 16, fill_dz, 0)

        base = s * SROWS
        for kk in range(SROWS // ZROWS):
            pltpu.sync_copy(zbuf, acc.at[pl.ds(base + kk * ZROWS, ZROWS)])
        plsc.subcore_barrier()

        onev = jnp.full((16,), 1.0, jnp.float32)
        ebase = wid * EPW

        def idx_start(j):
            cj = jnp.minimum(j, NCHUNK - 1)
            off = ebase + cj * CHUNK
            slot = cj & 3
            pltpu.async_copy(ei_hbm.at[0, pl.ds(off, CHUNK)], sb.at[slot], semi1)
            pltpu.async_copy(ei_hbm.at[1, pl.ds(off, CHUNK)], db.at[slot], semi2)

        def idx_wait(j):
            slot = j & 3
            pltpu.make_async_copy(ei_hbm.at[0, pl.ds(0, CHUNK)], sb.at[slot], semi1).wait()
            pltpu.make_async_copy(ei_hbm.at[1, pl.ds(0, CHUNK)], db.at[slot], semi2).wait()

        def scatter_deg(slot):
            for t in range(CHUNK // 16):
                plsc.addupdate_scatter(degp, [db[slot, pl.ds(t * 16, 16)]], onev)

        def halfstep(j_next, buf_cur, sem_cur, buf_nxt, sem_nxt):
            idx_start(j_next + 1)
            idx_wait(j_next)
            slot_nxt = j_next & 3
            slot_cur = (j_next + 3) & 3
            pltpu.async_copy(feat_hbm.at[sb.at[slot_nxt]], buf_nxt, sem_nxt)
            pltpu.make_async_copy(feat_hbm.at[sb.at[slot_cur]], buf_cur, sem_cur).wait()
            scatter_deg(slot_cur)
            pltpu.sync_copy(buf_cur, acc.at[db.at[slot_cur]], add=True)

        idx_start(0)
        idx_start(1)
        idx_wait(0)
        pltpu.async_copy(feat_hbm.at[sb.at[0]], abuf, sem1)

        def pair_body(i, _):
            halfstep(2 * i + 1, abuf, sem1, bbuf, sem2)
            halfstep(2 * i + 2, bbuf, sem2, abuf, sem1)
            return 0
        lax.fori_loop(0, (NCHUNK - 1) // 2, pair_body, 0)

        lslot = (NCHUNK - 1) % 4
        idx_wait(NCHUNK - 1)
        pltpu.make_async_copy(feat_hbm.at[sb.at[lslot]], abuf, sem1).wait()
        scatter_deg(lslot)
        pltpu.sync_copy(abuf, acc.at[db.at[lslot]], add=True)

        plsc.subcore_barrier()

        pltpu.sync_copy(acc.at[pl.ds(base, SROWS)],
                        out_hbm.at[c, pl.ds(base, SROWS)])
        pltpu.sync_copy(degp, deg_hbm.at[wid])

    return k(feature, ei)


def _tc_finish_kernel(parts, degp, feature, wt, b2):
    R = 1024

    def body(p_ref, d_ref, f_ref, wt_ref, b_ref, o_ref):
        ssum = p_ref[0] + p_ref[1]
        dsum = jnp.sum(d_ref[...], axis=0)[:, None]
        x = (f_ref[...] * ssum) / jnp.maximum(dsum, 1.0)
        o_ref[...] = (
            jnp.dot(x, wt_ref[...], preferred_element_type=jnp.float32) + b_ref[...]
        )

    return pl.pallas_call(
        body,
        grid=((N_NODES + R - 1) // R,),
        in_specs=[
            pl.BlockSpec((NC, R, D), lambda i: (0, i, 0)),
            pl.BlockSpec((NW, R), lambda i: (0, i)),
            pl.BlockSpec((R, D), lambda i: (i, 0)),
            pl.BlockSpec((D, D), lambda i: (0, 0)),
            pl.BlockSpec((1, D), lambda i: (0, 0)),
        ],
        out_specs=pl.BlockSpec((R, D), lambda i: (i, 0)),
        out_shape=jax.ShapeDtypeStruct((N_NODES, D), jnp.float32),
    )(parts, degp, feature, wt, b2)


def kernel(feature, edge_index, W, b):
    ei = edge_index.astype(jnp.int32)
    parts, degp = _sc_edge_kernel(feature, ei)
    return _tc_finish_kernel(parts, degp, feature, W.T, b.reshape(1, D))


# speedup vs baseline: 22.5804x; 1.1658x over previous
import functools

import jax
import jax.numpy as jnp
from jax import lax
from jax.experimental import pallas as pl
from jax.experimental.pallas import tpu as pltpu
from jax.experimental.pallas import tpu_sc as plsc

N_NODES = 10000
N_EDGES = 320000
D = 128

NC = 2
NS = 16
NW = NC * NS
EPW = N_EDGES // NW
CHUNK = 80
NCHUNK = EPW // CHUNK
N_PAD = 10240
SROWS = N_PAD // NS
ZROWS = 8
IDX_SHIFT = 14


def _sc_edge_kernel(feature, ei):
    mesh = plsc.VectorSubcoreMesh(
        core_axis_name="c", subcore_axis_name="s", num_cores=NC, num_subcores=NS
    )

    @functools.partial(
        pl.kernel,
        mesh=mesh,
        compiler_params=pltpu.CompilerParams(use_tc_tiling_on_sc=False, needs_layout_passes=False),
        out_type=(
            jax.ShapeDtypeStruct((NC, N_PAD, D), jnp.float32),
            jax.ShapeDtypeStruct((NW, N_PAD), jnp.float32),
        ),
        scratch_types=[
            pltpu.VMEM_SHARED((N_PAD, D), jnp.float32),
            pltpu.VMEM((8, CHUNK), jnp.int32),
            pltpu.VMEM((8, CHUNK), jnp.int32),
            pltpu.VMEM((CHUNK, D), jnp.float32),
            pltpu.VMEM((CHUNK, D), jnp.float32),
            pltpu.VMEM((CHUNK, D), jnp.float32),
            pltpu.VMEM((ZROWS, D), jnp.float32),
            pltpu.VMEM((N_PAD,), jnp.float32),
            pltpu.SemaphoreType.DMA,
            pltpu.SemaphoreType.DMA,
            pltpu.SemaphoreType.DMA,
            pltpu.SemaphoreType.DMA,
            pltpu.SemaphoreType.DMA,
            pltpu.SemaphoreType.DMA,
            pltpu.SemaphoreType.DMA,
            pltpu.SemaphoreType.DMA,
        ],
    )
    def k(feat_hbm, ei_hbm, out_hbm, deg_hbm,
          acc, sb, db, b0, b1, b2, zbuf, degp,
          g0, g1, g2, s0, s1, s2, semi1, semi2):
        c = lax.axis_index("c")
        s = lax.axis_index("s")
        wid = c * NS + s

        zv = jnp.zeros((16,), jnp.float32)

        def fill_z(r, _):
            for t in range(D // 16):
                zbuf[r, pl.ds(t * 16, 16)] = zv
            return 0
        lax.fori_loop(0, ZROWS, fill_z, 0)

        def fill_dz(r, _):
            degp[pl.ds(r * 16, 16)] = zv
            return 0
        lax.fori_loop(0, N_PAD // 16, fill_dz, 0)

        base = s * SROWS
        for kk in range(SROWS // ZROWS):
            pltpu.sync_copy(zbuf, acc.at[pl.ds(base + kk * ZROWS, ZROWS)])
        plsc.subcore_barrier()

        onev = jnp.full((16,), 1.0, jnp.float32)
        ebase = wid * EPW

        def idx_start(j):
            cj = jnp.minimum(j, NCHUNK - 1)
            off = ebase + cj * CHUNK
            slot = cj & 7
            pltpu.async_copy(ei_hbm.at[0, pl.ds(off, CHUNK)], sb.at[slot], semi1)
            pltpu.async_copy(ei_hbm.at[1, pl.ds(off, CHUNK)], db.at[slot], semi2)

        def idx_wait(j):
            slot = j & 7
            pltpu.make_async_copy(ei_hbm.at[0, pl.ds(0, CHUNK)], sb.at[slot], semi1).wait()
            pltpu.make_async_copy(ei_hbm.at[1, pl.ds(0, CHUNK)], db.at[slot], semi2).wait()

        def scatter_deg(slot):
            for t in range(CHUNK // 16):
                plsc.addupdate_scatter(degp, [db[slot, pl.ds(t * 16, 16)]], onev)

        def gather_start(j, buf, gsem):
            pltpu.async_copy(feat_hbm.at[sb.at[j & 7]], buf, gsem)

        def gather_wait(j, buf, gsem):
            pltpu.make_async_copy(feat_hbm.at[sb.at[j & 7]], buf, gsem).wait()

        def scat_start(j, buf, ssem):
            pltpu.async_copy(buf, acc.at[db.at[j & 7]], ssem, add=True)

        def scat_wait(j, buf, ssem):
            pltpu.make_async_copy(buf, acc.at[db.at[j & 7]], ssem).wait()

        def step(j, bj, gj, sj, bp, gp, sp, wait_scat):
            idx_wait(j)
            if wait_scat:
                scat_wait(j - 3, bj, sj)
            gather_start(j, bj, gj)
            gather_wait(j - 2, bp, gp)
            scatter_deg((j - 2) & 7)
            scat_start(j - 2, bp, sp)
            idx_start(j + 4)

        for jj in range(6):
            idx_start(jj)
        idx_wait(0)
        gather_start(0, b0, g0)
        idx_wait(1)
        gather_start(1, b1, g1)

        step(2, b2, g2, s2, b0, g0, s0, False)
        step(3, b0, g0, s0, b1, g1, s1, True)
        step(4, b1, g1, s1, b2, g2, s2, True)

        def triple(i, _):
            j = 5 + 3 * i
            step(j, b2, g2, s2, b0, g0, s0, True)
            step(j + 1, b0, g0, s0, b1, g1, s1, True)
            step(j + 2, b1, g1, s1, b2, g2, s2, True)
            return 0
        lax.fori_loop(0, (NCHUNK - 5) // 3, triple, 0)

        gather_wait(NCHUNK - 2, b0, g0)
        scatter_deg((NCHUNK - 2) & 7)
        scat_start(NCHUNK - 2, b0, s0)
        gather_wait(NCHUNK - 1, b1, g1)
        scatter_deg((NCHUNK - 1) & 7)
        scat_start(NCHUNK - 1, b1, s1)
        scat_wait(NCHUNK - 3, b2, s2)
        scat_wait(NCHUNK - 2, b0, s0)
        scat_wait(NCHUNK - 1, b1, s1)
        for _ in range(4):
            idx_wait(0)

        plsc.subcore_barrier()

        pltpu.sync_copy(acc.at[pl.ds(base, SROWS)],
                        out_hbm.at[c, pl.ds(base, SROWS)])
        pltpu.sync_copy(degp, deg_hbm.at[wid])

    return k(feature, ei)


def _tc_finish_kernel(parts, degp, feature, wt, b2):
    R = 1024

    def body(p_ref, d_ref, f_ref, wt_ref, b_ref, o_ref):
        ssum = p_ref[0] + p_ref[1]
        dsum = jnp.sum(d_ref[...], axis=0)[:, None]
        x = (f_ref[...] * ssum) / jnp.maximum(dsum, 1.0)
        o_ref[...] = (
            jnp.dot(x, wt_ref[...], preferred_element_type=jnp.float32) + b_ref[...]
        )

    return pl.pallas_call(
        body,
        grid=((N_NODES + R - 1) // R,),
        in_specs=[
            pl.BlockSpec((NC, R, D), lambda i: (0, i, 0)),
            pl.BlockSpec((NW, R), lambda i: (0, i)),
            pl.BlockSpec((R, D), lambda i: (i, 0)),
            pl.BlockSpec((D, D), lambda i: (0, 0)),
            pl.BlockSpec((1, D), lambda i: (0, 0)),
        ],
        out_specs=pl.BlockSpec((R, D), lambda i: (i, 0)),
        out_shape=jax.ShapeDtypeStruct((N_NODES, D), jnp.float32),
    )(parts, degp, feature, wt, b2)


def kernel(feature, edge_index, W, b):
    ei = edge_index.astype(jnp.int32)
    parts, degp = _sc_edge_kernel(feature, ei)
    return _tc_finish_kernel(parts, degp, feature, W.T, b.reshape(1, D))


# speedup vs baseline: 22.6368x; 1.0025x over previous
import functools

import jax
import jax.numpy as jnp
from jax import lax
from jax.experimental import pallas as pl
from jax.experimental.pallas import tpu as pltpu
from jax.experimental.pallas import tpu_sc as plsc

N_NODES = 10000
N_EDGES = 320000
D = 128

NC = 2
NS = 16
NW = NC * NS
EPW = N_EDGES // NW
CHUNK = 80
NCHUNK = EPW // CHUNK
N_PAD = 10240
SROWS = N_PAD // NS
ZROWS = 8
IDX_SHIFT = 14


def _sc_edge_kernel(feature, ei):
    mesh = plsc.VectorSubcoreMesh(
        core_axis_name="c", subcore_axis_name="s", num_cores=NC, num_subcores=NS
    )

    @functools.partial(
        pl.kernel,
        mesh=mesh,
        compiler_params=pltpu.CompilerParams(use_tc_tiling_on_sc=False, needs_layout_passes=False),
        out_type=(
            jax.ShapeDtypeStruct((NC, N_PAD, D), jnp.float32),
            jax.ShapeDtypeStruct((NW, N_PAD), jnp.float32),
        ),
        scratch_types=[
            pltpu.VMEM_SHARED((N_PAD, D), jnp.float32),
            pltpu.VMEM((8, CHUNK), jnp.int32),
            pltpu.VMEM((8, CHUNK), jnp.int32),
            pltpu.VMEM((CHUNK, D), jnp.float32),
            pltpu.VMEM((CHUNK, D), jnp.float32),
            pltpu.VMEM((CHUNK, D), jnp.float32),
            pltpu.VMEM((ZROWS, D), jnp.float32),
            pltpu.VMEM((N_PAD,), jnp.float32),
            pltpu.SemaphoreType.DMA,
            pltpu.SemaphoreType.DMA,
            pltpu.SemaphoreType.DMA,
            pltpu.SemaphoreType.DMA,
            pltpu.SemaphoreType.DMA,
            pltpu.SemaphoreType.DMA,
            pltpu.SemaphoreType.DMA,
            pltpu.SemaphoreType.DMA,
        ],
    )
    def k(feat_hbm, ei_hbm, out_hbm, deg_hbm,
          acc, sb, db, b0, b1, b2, zbuf, degp,
          g0, g1, g2, s0, s1, s2, semi1, semi2):
        c = lax.axis_index("c")
        s = lax.axis_index("s")
        wid = c * NS + s

        zv = jnp.zeros((16,), jnp.float32)

        def fill_z(r, _):
            for t in range(D // 16):
                zbuf[r, pl.ds(t * 16, 16)] = zv
            return 0
        lax.fori_loop(0, ZROWS, fill_z, 0)

        def fill_dz(r, _):
            degp[pl.ds(r * 16, 16)] = zv
            return 0
        lax.fori_loop(0, N_PAD // 16, fill_dz, 0)

        base = s * SROWS
        for kk in range(SROWS // ZROWS):
            pltpu.sync_copy(zbuf, acc.at[pl.ds(base + kk * ZROWS, ZROWS)])
        plsc.subcore_barrier()

        onev = jnp.full((16,), 1.0, jnp.float32)
        ebase = wid * EPW

        def idx_start(j):
            cj = jnp.minimum(j, NCHUNK - 1)
            off = ebase + cj * CHUNK
            slot = cj & 7
            pltpu.async_copy(ei_hbm.at[0, pl.ds(off, CHUNK)], sb.at[slot], semi1)
            pltpu.async_copy(ei_hbm.at[1, pl.ds(off, CHUNK)], db.at[slot], semi2)

        def idx_wait(j):
            slot = j & 7
            pltpu.make_async_copy(ei_hbm.at[0, pl.ds(0, CHUNK)], sb.at[slot], semi1).wait()
            pltpu.make_async_copy(ei_hbm.at[1, pl.ds(0, CHUNK)], db.at[slot], semi2).wait()

        def scatter_deg(slot):
            for t in range(CHUNK // 16):
                plsc.addupdate_scatter(degp, [db[slot, pl.ds(t * 16, 16)]], onev)

        H = CHUNK // 2

        def gather_start(j, buf, gsem):
            slot = j & 7
            pltpu.async_copy(feat_hbm.at[sb.at[slot, pl.ds(0, H)]],
                             buf.at[pl.ds(0, H)], gsem)
            pltpu.async_copy(feat_hbm.at[sb.at[slot, pl.ds(H, H)]],
                             buf.at[pl.ds(H, H)], gsem)

        def gather_wait(j, buf, gsem):
            slot = j & 7
            pltpu.make_async_copy(feat_hbm.at[sb.at[slot, pl.ds(0, H)]],
                                  buf.at[pl.ds(0, H)], gsem).wait()
            pltpu.make_async_copy(feat_hbm.at[sb.at[slot, pl.ds(H, H)]],
                                  buf.at[pl.ds(H, H)], gsem).wait()

        def scat_start(j, buf, ssem):
            pltpu.async_copy(buf, acc.at[db.at[j & 7]], ssem, add=True)

        def scat_wait(j, buf, ssem):
            pltpu.make_async_copy(buf, acc.at[db.at[j & 7]], ssem).wait()

        def step(j, bj, gj, sj, bp, gp, sp, wait_scat):
            idx_wait(j)
            if wait_scat:
                scat_wait(j - 3, bj, sj)
            gather_start(j, bj, gj)
            gather_wait(j - 2, bp, gp)
            scatter_deg((j - 2) & 7)
            scat_start(j - 2, bp, sp)
            idx_start(j + 4)

        for jj in range(6):
            idx_start(jj)
        idx_wait(0)
        gather_start(0, b0, g0)
        idx_wait(1)
        gather_start(1, b1, g1)

        step(2, b2, g2, s2, b0, g0, s0, False)
        step(3, b0, g0, s0, b1, g1, s1, True)
        step(4, b1, g1, s1, b2, g2, s2, True)

        def triple(i, _):
            j = 5 + 3 * i
            step(j, b2, g2, s2, b0, g0, s0, True)
            step(j + 1, b0, g0, s0, b1, g1, s1, True)
            step(j + 2, b1, g1, s1, b2, g2, s2, True)
            return 0
        lax.fori_loop(0, (NCHUNK - 5) // 3, triple, 0)

        gather_wait(NCHUNK - 2, b0, g0)
        scatter_deg((NCHUNK - 2) & 7)
        scat_start(NCHUNK - 2, b0, s0)
        gather_wait(NCHUNK - 1, b1, g1)
        scatter_deg((NCHUNK - 1) & 7)
        scat_start(NCHUNK - 1, b1, s1)
        scat_wait(NCHUNK - 3, b2, s2)
        scat_wait(NCHUNK - 2, b0, s0)
        scat_wait(NCHUNK - 1, b1, s1)
        for _ in range(4):
            idx_wait(0)

        plsc.subcore_barrier()

        pltpu.sync_copy(acc.at[pl.ds(base, SROWS)],
                        out_hbm.at[c, pl.ds(base, SROWS)])
        pltpu.sync_copy(degp, deg_hbm.at[wid])

    return k(feature, ei)


def _tc_finish_kernel(parts, degp, feature, wt, b2):
    R = 1024

    def body(p_ref, d_ref, f_ref, wt_ref, b_ref, o_ref):
        ssum = p_ref[0] + p_ref[1]
        dsum = jnp.sum(d_ref[...], axis=0)[:, None]
        x = (f_ref[...] * ssum) / jnp.maximum(dsum, 1.0)
        o_ref[...] = (
            jnp.dot(x, wt_ref[...], preferred_element_type=jnp.float32) + b_ref[...]
        )

    return pl.pallas_call(
        body,
        grid=((N_NODES + R - 1) // R,),
        in_specs=[
            pl.BlockSpec((NC, R, D), lambda i: (0, i, 0)),
            pl.BlockSpec((NW, R), lambda i: (0, i)),
            pl.BlockSpec((R, D), lambda i: (i, 0)),
            pl.BlockSpec((D, D), lambda i: (0, 0)),
            pl.BlockSpec((1, D), lambda i: (0, 0)),
        ],
        out_specs=pl.BlockSpec((R, D), lambda i: (i, 0)),
        out_shape=jax.ShapeDtypeStruct((N_NODES, D), jnp.float32),
    )(parts, degp, feature, wt, b2)


def kernel(feature, edge_index, W, b):
    ei = edge_index.astype(jnp.int32)
    parts, degp = _sc_edge_kernel(feature, ei)
    return _tc_finish_kernel(parts, degp, feature, W.T, b.reshape(1, D))
